# baseline (speedup 1.0000x reference)
import functools

import jax
import jax.numpy as jnp
from jax import lax
from jax.experimental import pallas as pl
from jax.experimental.pallas import tpu as pltpu
from jax.experimental.pallas import tpu_sc as plsc

N = 10000
D = 256
E = 160000
L = 40

NC = 2
NS = 16
NW = NC * NS

E_PAD = 163840
ECH = E_PAD // NW
GCH = ECH // 16

OWN = 320
N_PAD = NW * OWN
ACC_ROWS = 328
DUMP = 320
RSIZE = 128
KD = D // 16

RUN_Q = 32
BCAP = 199680
STG = 5376

_MESH = plsc.VectorSubcoreMesh(
    core_axis_name="c", subcore_axis_name="s", num_cores=NC, num_subcores=NS)
_NOLAYOUT = pltpu.CompilerParams(needs_layout_passes=False)


def _sc_count_body(dst_hbm, counts_out, dst_blk, hist):
  c = lax.axis_index("c")
  s = lax.axis_index("s")
  w = c * NS + s

  z16 = jnp.zeros((16,), jnp.float32)
  ones16 = jnp.ones((16,), jnp.float32)
  for i in range(3):
    hist[pl.ds(i * 16, 16)] = z16

  pltpu.sync_copy(dst_hbm.at[pl.ds(w * ECH, ECH)], dst_blk)

  def grp(j, carry):
    dvec = dst_blk[pl.ds(j * 16, 16)]
    ow = ((dvec >> 6) * 3277) >> 14
    ow = jnp.where(dvec >= 0, ow, 32)
    plsc.addupdate_scatter(hist, [ow], ones16)
    return carry
  lax.fori_loop(0, GCH, grp, 0)

  pltpu.sync_copy(hist, counts_out.at[pl.ds(w * 48, 48)])


_sc_count = pl.kernel(
    _sc_count_body,
    out_type=(jax.ShapeDtypeStruct((NW * 48,), jnp.float32),),
    mesh=_MESH,
    scratch_types=(
        pltpu.VMEM((ECH,), jnp.int32),
        pltpu.VMEM((48,), jnp.float32),
    ),
    compiler_params=_NOLAYOUT)


def _prefix_body(cnt_ref, offs_ref, pr_ref, rounds_ref, starts_ref):
  cnt = cnt_ref[...][:, :32]
  pr = jnp.floor((cnt + 31.0) / 32.0) * 32.0
  tot = jnp.sum(pr, axis=0)
  extra = 128.0 * jnp.ceil(tot / 128.0) - tot
  rio = lax.broadcasted_iota(jnp.int32, (32, 32), 0)
  cio = lax.broadcasted_iota(jnp.int32, (32, 32), 1)
  pr = pr + jnp.where(rio == 31, extra[None, :], 0.0)
  tot = tot + extra
  lstrict = (rio > cio).astype(jnp.float32)
  starts = jnp.dot(lstrict, tot[:, None],
                   preferred_element_type=jnp.float32)[:, 0]
  offs = starts[None, :] + jnp.dot(lstrict, pr,
                                   preferred_element_type=jnp.float32)
  offs_ref[...] = offs.astype(jnp.int32)
  pr_ref[...] = pr.astype(jnp.int32)
  rounds_ref[...] = (tot * (1.0 / 128.0)).astype(jnp.int32)
  starts_ref[...] = starts.astype(jnp.int32)


def _tc_prefix(counts2d):
  return pl.pallas_call(
      _prefix_body,
      out_shape=(jax.ShapeDtypeStruct((32, 32), jnp.int32),
                 jax.ShapeDtypeStruct((32, 32), jnp.int32),
                 jax.ShapeDtypeStruct((32,), jnp.int32),
                 jax.ShapeDtypeStruct((32,), jnp.int32)),
  )(counts2d)


def _sc_bucket_body(src_hbm, dst_hbm, offs_hbm, pr_hbm, bsrc_out, bldst_out,
                    src_blk, dst_blk, stage_s, stage_d, offv, prv):
  c = lax.axis_index("c")
  s = lax.axis_index("s")
  w = c * NS + s

  zi16 = jnp.zeros((16,), jnp.int32)
  dump16 = jnp.full((16,), DUMP, jnp.int32)

  pltpu.sync_copy(src_hbm.at[pl.ds(w * ECH, ECH)], src_blk)
  pltpu.sync_copy(dst_hbm.at[pl.ds(w * ECH, ECH)], dst_blk)
  pltpu.sync_copy(offs_hbm, offv)
  pltpu.sync_copy(pr_hbm, prv)

  for o in range(32):
    olo = o * OWN

    def grp(j, cnt):
      dvec = dst_blk[pl.ds(j * 16, 16)]
      svec = src_blk[pl.ds(j * 16, 16)]
      m = (dvec >= olo) & (dvec < olo + OWN)
      plsc.store_compressed(stage_s.at[pl.ds(cnt, 16)], svec, mask=m)
      plsc.store_compressed(stage_d.at[pl.ds(cnt, 16)], dvec - olo, mask=m)
      return cnt + jnp.sum(m.astype(jnp.int32))
    cnt = lax.fori_loop(0, GCH, grp, 0)

    lofs = offv[pl.ds(w * 32 + (o // 16) * 16, 16)]
    lpr = prv[pl.ds(w * 32 + (o // 16) * 16, 16)]
    off_o = pl.multiple_of(lofs[o % 16], RUN_Q)
    pr_o = lpr[o % 16]

    npg = (pr_o - cnt + 15) // 16

    def padg(i, carry):
      stage_s[pl.ds(cnt + i * 16, 16)] = zi16
      stage_d[pl.ds(cnt + i * 16, 16)] = dump16
      return carry
    lax.fori_loop(0, npg, padg, 0)

    nch = pr_o // RUN_Q

    def dmac(i, carry):
      pltpu.sync_copy(stage_s.at[pl.ds(i * RUN_Q, RUN_Q)],
                      bsrc_out.at[pl.ds(off_o + i * RUN_Q, RUN_Q)])
      pltpu.sync_copy(stage_d.at[pl.ds(i * RUN_Q, RUN_Q)],
                      bldst_out.at[pl.ds(off_o + i * RUN_Q, RUN_Q)])
      return carry
    lax.fori_loop(0, nch, dmac, 0)


_sc_bucket = pl.kernel(
    _sc_bucket_body,
    out_type=(jax.ShapeDtypeStruct((BCAP,), jnp.int32),
              jax.ShapeDtypeStruct((BCAP,), jnp.int32)),
    mesh=_MESH,
    scratch_types=(
        pltpu.VMEM((ECH,), jnp.int32),
        pltpu.VMEM((ECH,), jnp.int32),
        pltpu.VMEM((STG,), jnp.int32),
        pltpu.VMEM((STG,), jnp.int32),
        pltpu.VMEM((NW * 32,), jnp.int32),
        pltpu.VMEM((NW * 32,), jnp.int32),
    ),
    compiler_params=_NOLAYOUT)


def _sc_agg_body(bsrc, bldst, h_hbm, rounds_hbm, starts_hbm, agg_out,
                 gidx0, gidx1, sdx0, sdx1, rows0, rows1, acc, deg,
                 bsv, blv, rv, sv, sem0, sem1):
  c = lax.axis_index("c")
  s = lax.axis_index("s")
  w = c * NS + s
  wlo = w * OWN

  z16 = jnp.zeros((16,), jnp.float32)
  ones16 = jnp.ones((16,), jnp.float32)

  def zacc(i, carry):
    for k in range(KD):
      acc[i, pl.ds(k * 16, 16)] = z16
    return carry
  lax.fori_loop(0, ACC_ROWS, zacc, 0)
  def zdeg(i, carry):
    deg[pl.ds(i * 16, 16)] = z16
    return carry
  lax.fori_loop(0, ACC_ROWS // 8, zdeg, 0)

  pltpu.sync_copy(rounds_hbm, rv)
  pltpu.sync_copy(starts_hbm, sv)
  lanes = jnp.arange(16, dtype=jnp.int32)
  msk = lanes == s
  zi = jnp.zeros((16,), jnp.int32)
  nr = jnp.sum(jnp.where(msk, rv[pl.ds(c * 16, 16)], zi))
  st = pl.multiple_of(jnp.sum(jnp.where(msk, sv[pl.ds(c * 16, 16)], zi)),
                      RSIZE)

  def accumulate(sdx_b, rows_b):
    for j in range(4):
      plsc.addupdate_scatter(deg, [sdx_b[pl.ds(j * 16, 16)]], ones16)

    def acc_grp(i16, carry2):
      lvec = sdx_b[pl.ds(i16 * 16, 16)]
      base = i16 * 16
      for lane in range(16):
        r_own = lvec[lane]
        for k in range(KD):
          acc[r_own, pl.ds(k * 16, 16)] = (
              acc[r_own, pl.ds(k * 16, 16)]
              + rows_b[base + lane, pl.ds(k * 16, 16)])
      return carry2
    lax.fori_loop(0, 4, acc_grp, 0)

  def stage(gidx_b, sdx_b, off):
    for q in range(4):
      gidx_b[pl.ds(q * 16, 16)] = bsv[pl.ds(off + q * 16, 16)]
      sdx_b[pl.ds(q * 16, 16)] = blv[pl.ds(off + q * 16, 16)]

  def pair(p, carry):
    @pl.when((p & 7) == 0)
    def _():
      pltpu.sync_copy(bsrc.at[pl.ds(st + p * 128, 1024)], bsv)
      pltpu.sync_copy(bldst.at[pl.ds(st + p * 128, 1024)], blv)

    off0 = (p & 7) * 128
    stage(gidx0, sdx0, off0)
    cp0 = pltpu.async_copy(h_hbm.at[gidx0], rows0, sem0)

    @pl.when(p > 0)
    def _():
      pltpu.make_async_copy(h_hbm.at[gidx1], rows1, sem1).wait()
      accumulate(sdx1, rows1)

    stage(gidx1, sdx1, off0 + 64)
    pltpu.async_copy(h_hbm.at[gidx1], rows1, sem1)

    cp0.wait()
    accumulate(sdx0, rows0)
    return carry
  lax.fori_loop(0, nr, pair, 0)

  @pl.when(nr > 0)
  def _():
    pltpu.make_async_copy(h_hbm.at[gidx1], rows1, sem1).wait()
    accumulate(sdx1, rows1)

  def norm(i16, carry):
    dv = jnp.maximum(deg[pl.ds(i16 * 16, 16)], 1.0)
    rinv = ones16 / dv
    for lane in range(16):
      r_own = i16 * 16 + lane
      dsp = rinv[lane] * ones16
      for k in range(KD):
        acc[r_own, pl.ds(k * 16, 16)] = acc[r_own, pl.ds(k * 16, 16)] * dsp
    return carry
  lax.fori_loop(0, OWN // 16, norm, 0)

  pltpu.sync_copy(acc.at[pl.ds(0, OWN)], agg_out.at[pl.ds(wlo, OWN)])


_sc_agg = pl.kernel(
    _sc_agg_body,
    out_type=(jax.ShapeDtypeStruct((N_PAD, D), jnp.float32),),
    mesh=_MESH,
    scratch_types=(
        pltpu.VMEM((64,), jnp.int32),
        pltpu.VMEM((64,), jnp.int32),
        pltpu.VMEM((64,), jnp.int32),
        pltpu.VMEM((64,), jnp.int32),
        pltpu.VMEM((64, D), jnp.float32),
        pltpu.VMEM((64, D), jnp.float32),
        pltpu.VMEM((ACC_ROWS, D), jnp.float32),
        pltpu.VMEM((ACC_ROWS,), jnp.float32),
        pltpu.VMEM((1024,), jnp.int32),
        pltpu.VMEM((1024,), jnp.int32),
        pltpu.VMEM((48,), jnp.int32),
        pltpu.VMEM((48,), jnp.int32),
        pltpu.SemaphoreType.DMA,
        pltpu.SemaphoreType.DMA,
    ),
    compiler_params=_NOLAYOUT)


def _mm_body(x_ref, w_ref, o_ref):
  o_ref[...] = jnp.dot(x_ref[...], w_ref[...],
                       preferred_element_type=jnp.float32)


def _tc_matmul(x, w):
  return pl.pallas_call(
      _mm_body,
      out_shape=jax.ShapeDtypeStruct((x.shape[0], w.shape[1]), jnp.float32),
  )(x, w)


def _mid_body(agg_ref, b_ref, g_ref, be_ref, w_ref, h_ref):
  x = agg_ref[:N, :] + b_ref[...]
  mu = jnp.mean(x, axis=0, keepdims=True)
  var = jnp.mean((x - mu) ** 2, axis=0, keepdims=True)
  x = (x - mu) * lax.rsqrt(var + 1e-5) * g_ref[...] + be_ref[...]
  x = jnp.where(x > 0, x, jnp.exp(x) - 1.0)
  h_ref[...] = jnp.dot(x, w_ref[...], preferred_element_type=jnp.float32)


def _tc_mid(agg, b, gamma, beta, w):
  return pl.pallas_call(
      _mid_body,
      out_shape=jax.ShapeDtypeStruct((N, D), jnp.float32),
  )(agg, b, gamma, beta, w)


def _final_body(agg_ref, b_ref, wl_ref, bl_ref, x_ref, lg_ref):
  x = agg_ref[:N, :] + b_ref[...]
  x_ref[...] = x
  lg_ref[...] = (jnp.dot(x, wl_ref[...], preferred_element_type=jnp.float32)
                 + bl_ref[...])


def _tc_final(agg, b, wl, bl):
  return pl.pallas_call(
      _final_body,
      out_shape=(jax.ShapeDtypeStruct((N, D), jnp.float32),
                 jax.ShapeDtypeStruct((N, L), jnp.float32)),
  )(agg, b, wl, bl)


def kernel(emb, W1, b1, W2, b2, W3, b3, gamma, beta, Wl, bl, edge_index):
  src = edge_index[0].astype(jnp.int32)
  dst = edge_index[1].astype(jnp.int32)
  npad = E_PAD - src.shape[0]
  src_p = jnp.concatenate([src, jnp.zeros((npad,), jnp.int32)])
  dst_p = jnp.concatenate([dst, jnp.full((npad,), -1, jnp.int32)])

  (counts,) = _sc_count(dst_p)
  offs, pr, rounds, starts = _tc_prefix(counts.reshape(NW, 48))
  bsrc, bldst = _sc_bucket(src_p, dst_p, offs.reshape(NW * 32),
                           pr.reshape(NW * 32))
  rounds48 = jnp.pad(rounds, (0, 16))
  starts48 = jnp.pad(starts, (0, 16))

  h1 = _tc_matmul(emb, W1)
  (agg1,) = _sc_agg(bsrc, bldst, h1, rounds48, starts48)
  h2 = _tc_mid(agg1, b1, gamma, beta, W2)
  (agg2,) = _sc_agg(bsrc, bldst, h2, rounds48, starts48)
  h3 = _tc_mid(agg2, b2, gamma, beta, W3)
  (agg3,) = _sc_agg(bsrc, bldst, h3, rounds48, starts48)
  x3, logits = _tc_final(agg3, b3, Wl, bl)
  return (x3, logits)


# speedup vs baseline: 1.4440x; 1.4440x over previous
import functools

import jax
import jax.numpy as jnp
from jax import lax
from jax.experimental import pallas as pl
from jax.experimental.pallas import tpu as pltpu
from jax.experimental.pallas import tpu_sc as plsc

N = 10000
D = 256
E = 160000
L = 40

NC = 2
NS = 16
NW = NC * NS

E_PAD = 163840
ECH = E_PAD // NW
GCH = ECH // 16

OWN = 320
N_PAD = NW * OWN
ACC_ROWS = 328
DUMP = 320
RSIZE = 128
KD = D // 16

RUN_Q = 8
BCAP = 199680
STG = 5376

_MESH = plsc.VectorSubcoreMesh(
    core_axis_name="c", subcore_axis_name="s", num_cores=NC, num_subcores=NS)
_NOLAYOUT = pltpu.CompilerParams(needs_layout_passes=False)


def _sc_count_body(dst_hbm, counts_out, dst_blk, hist):
  c = lax.axis_index("c")
  s = lax.axis_index("s")
  w = c * NS + s

  z16 = jnp.zeros((16,), jnp.float32)
  ones16 = jnp.ones((16,), jnp.float32)
  for i in range(3):
    hist[pl.ds(i * 16, 16)] = z16

  pltpu.sync_copy(dst_hbm.at[pl.ds(w * ECH, ECH)], dst_blk)

  def grp(j, carry):
    dvec = dst_blk[pl.ds(j * 16, 16)]
    ow = ((dvec >> 6) * 3277) >> 14
    ow = jnp.where(dvec >= 0, ow, 32)
    plsc.addupdate_scatter(hist, [ow], ones16)
    return carry
  lax.fori_loop(0, GCH, grp, 0)

  pltpu.sync_copy(hist, counts_out.at[pl.ds(w * 48, 48)])


_sc_count = pl.kernel(
    _sc_count_body,
    out_type=(jax.ShapeDtypeStruct((NW * 48,), jnp.float32),),
    mesh=_MESH,
    scratch_types=(
        pltpu.VMEM((ECH,), jnp.int32),
        pltpu.VMEM((48,), jnp.float32),
    ),
    compiler_params=_NOLAYOUT)


def _prefix_body(cnt_ref, offs_ref, pr_ref, rounds_ref, starts_ref):
  cnt = cnt_ref[...][:, :32]
  pr = jnp.floor((cnt + 7.0) / 8.0) * 8.0
  tot = jnp.sum(pr, axis=0)
  extra = 128.0 * jnp.ceil(tot / 128.0) - tot
  rio = lax.broadcasted_iota(jnp.int32, (32, 32), 0)
  cio = lax.broadcasted_iota(jnp.int32, (32, 32), 1)
  pr = pr + jnp.where(rio == 31, extra[None, :], 0.0)
  tot = tot + extra
  lstrict = (rio > cio).astype(jnp.float32)
  starts = jnp.dot(lstrict, tot[:, None],
                   preferred_element_type=jnp.float32)[:, 0]
  offs = starts[None, :] + jnp.dot(lstrict, pr,
                                   preferred_element_type=jnp.float32)
  offs_ref[...] = offs.astype(jnp.int32)
  pr_ref[...] = pr.astype(jnp.int32)
  rounds_ref[...] = (tot * (1.0 / 128.0)).astype(jnp.int32)
  starts_ref[...] = starts.astype(jnp.int32)


def _tc_prefix(counts2d):
  return pl.pallas_call(
      _prefix_body,
      out_shape=(jax.ShapeDtypeStruct((32, 32), jnp.int32),
                 jax.ShapeDtypeStruct((32, 32), jnp.int32),
                 jax.ShapeDtypeStruct((32,), jnp.int32),
                 jax.ShapeDtypeStruct((32,), jnp.int32)),
  )(counts2d)


def _sc_bucket_body(src_hbm, dst_hbm, offs_hbm, pr_hbm, bsrc_out, bldst_out,
                    src_blk, dst_blk, stage_s, stage_d, offv, prv):
  c = lax.axis_index("c")
  s = lax.axis_index("s")
  w = c * NS + s

  zi16 = jnp.zeros((16,), jnp.int32)
  dump16 = jnp.full((16,), DUMP, jnp.int32)

  pltpu.sync_copy(src_hbm.at[pl.ds(w * ECH, ECH)], src_blk)
  pltpu.sync_copy(dst_hbm.at[pl.ds(w * ECH, ECH)], dst_blk)
  pltpu.sync_copy(offs_hbm, offv)
  pltpu.sync_copy(pr_hbm, prv)

  for o in range(32):
    olo = o * OWN

    def grp(j, cnt):
      dvec = dst_blk[pl.ds(j * 16, 16)]
      svec = src_blk[pl.ds(j * 16, 16)]
      m = (dvec >= olo) & (dvec < olo + OWN)
      plsc.store_compressed(stage_s.at[pl.ds(cnt, 16)], svec, mask=m)
      plsc.store_compressed(stage_d.at[pl.ds(cnt, 16)], dvec - olo, mask=m)
      return cnt + jnp.sum(m.astype(jnp.int32))
    cnt = lax.fori_loop(0, GCH, grp, 0)

    lofs = offv[pl.ds(w * 32 + (o // 16) * 16, 16)]
    lpr = prv[pl.ds(w * 32 + (o // 16) * 16, 16)]
    off_o = pl.multiple_of(lofs[o % 16], 8)
    pr_o = lpr[o % 16]

    npg = (pr_o - cnt + 15) // 16

    def padg(i, carry):
      stage_s[pl.ds(cnt + i * 16, 16)] = zi16
      stage_d[pl.ds(cnt + i * 16, 16)] = dump16
      return carry
    lax.fori_loop(0, npg, padg, 0)

    nch = pr_o // RUN_Q

    def dmac(i, carry):
      pltpu.sync_copy(stage_s.at[pl.ds(i * RUN_Q, RUN_Q)],
                      bsrc_out.at[pl.ds(off_o + i * RUN_Q, RUN_Q)])
      pltpu.sync_copy(stage_d.at[pl.ds(i * RUN_Q, RUN_Q)],
                      bldst_out.at[pl.ds(off_o + i * RUN_Q, RUN_Q)])
      return carry
    lax.fori_loop(0, nch, dmac, 0)


_sc_bucket = pl.kernel(
    _sc_bucket_body,
    out_type=(jax.ShapeDtypeStruct((BCAP,), jnp.int32),
              jax.ShapeDtypeStruct((BCAP,), jnp.int32)),
    mesh=_MESH,
    scratch_types=(
        pltpu.VMEM((ECH,), jnp.int32),
        pltpu.VMEM((ECH,), jnp.int32),
        pltpu.VMEM((STG,), jnp.int32),
        pltpu.VMEM((STG,), jnp.int32),
        pltpu.VMEM((NW * 32,), jnp.int32),
        pltpu.VMEM((NW * 32,), jnp.int32),
    ),
    compiler_params=_NOLAYOUT)


def _sc_agg_body(bsrc, bldst, h_hbm, rounds_hbm, starts_hbm, agg_out,
                 gidx0, gidx1, sdx0, sdx1, rows0, rows1, acc, deg,
                 bsv, blv, rv, sv, sem0, sem1):
  c = lax.axis_index("c")
  s = lax.axis_index("s")
  w = c * NS + s
  wlo = w * OWN

  z16 = jnp.zeros((16,), jnp.float32)
  ones16 = jnp.ones((16,), jnp.float32)

  def zacc(i, carry):
    for k in range(KD):
      acc[i, pl.ds(k * 16, 16)] = z16
    return carry
  lax.fori_loop(0, ACC_ROWS, zacc, 0)
  def zdeg(i, carry):
    deg[pl.ds(i * 16, 16)] = z16
    return carry
  lax.fori_loop(0, ACC_ROWS // 8, zdeg, 0)

  pltpu.sync_copy(rounds_hbm, rv)
  pltpu.sync_copy(starts_hbm, sv)
  lanes = jnp.arange(16, dtype=jnp.int32)
  msk = lanes == s
  zi = jnp.zeros((16,), jnp.int32)
  nr = jnp.sum(jnp.where(msk, rv[pl.ds(c * 16, 16)], zi))
  st = pl.multiple_of(jnp.sum(jnp.where(msk, sv[pl.ds(c * 16, 16)], zi)),
                      RSIZE)

  def accumulate(sdx_b, rows_b):
    for j in range(4):
      plsc.addupdate_scatter(deg, [sdx_b[pl.ds(j * 16, 16)]], ones16)

    def acc_grp(i16, carry2):
      lvec = sdx_b[pl.ds(i16 * 16, 16)]
      base = i16 * 16
      for lane in range(16):
        r_own = lvec[lane]
        for k in range(KD):
          acc[r_own, pl.ds(k * 16, 16)] = (
              acc[r_own, pl.ds(k * 16, 16)]
              + rows_b[base + lane, pl.ds(k * 16, 16)])
      return carry2
    lax.fori_loop(0, 4, acc_grp, 0)

  def stage(gidx_b, sdx_b, off):
    for q in range(4):
      gidx_b[pl.ds(q * 16, 16)] = bsv[pl.ds(off + q * 16, 16)]
      sdx_b[pl.ds(q * 16, 16)] = blv[pl.ds(off + q * 16, 16)]

  def pair(p, carry):
    @pl.when((p & 7) == 0)
    def _():
      pltpu.sync_copy(bsrc.at[pl.ds(st + p * 128, 1024)], bsv)
      pltpu.sync_copy(bldst.at[pl.ds(st + p * 128, 1024)], blv)

    off0 = (p & 7) * 128
    stage(gidx0, sdx0, off0)
    cp0 = pltpu.async_copy(h_hbm.at[gidx0], rows0, sem0)

    @pl.when(p > 0)
    def _():
      pltpu.make_async_copy(h_hbm.at[gidx1], rows1, sem1).wait()
      accumulate(sdx1, rows1)

    stage(gidx1, sdx1, off0 + 64)
    pltpu.async_copy(h_hbm.at[gidx1], rows1, sem1)

    cp0.wait()
    accumulate(sdx0, rows0)
    return carry
  lax.fori_loop(0, nr, pair, 0)

  @pl.when(nr > 0)
  def _():
    pltpu.make_async_copy(h_hbm.at[gidx1], rows1, sem1).wait()
    accumulate(sdx1, rows1)

  def norm(i16, carry):
    dv = jnp.maximum(deg[pl.ds(i16 * 16, 16)], 1.0)
    rinv = ones16 / dv
    for lane in range(16):
      r_own = i16 * 16 + lane
      dsp = rinv[lane] * ones16
      for k in range(KD):
        acc[r_own, pl.ds(k * 16, 16)] = acc[r_own, pl.ds(k * 16, 16)] * dsp
    return carry
  lax.fori_loop(0, OWN // 16, norm, 0)

  pltpu.sync_copy(acc.at[pl.ds(0, OWN)], agg_out.at[pl.ds(wlo, OWN)])


_sc_agg = pl.kernel(
    _sc_agg_body,
    out_type=(jax.ShapeDtypeStruct((N_PAD, D), jnp.float32),),
    mesh=_MESH,
    scratch_types=(
        pltpu.VMEM((64,), jnp.int32),
        pltpu.VMEM((64,), jnp.int32),
        pltpu.VMEM((64,), jnp.int32),
        pltpu.VMEM((64,), jnp.int32),
        pltpu.VMEM((64, D), jnp.float32),
        pltpu.VMEM((64, D), jnp.float32),
        pltpu.VMEM((ACC_ROWS, D), jnp.float32),
        pltpu.VMEM((ACC_ROWS,), jnp.float32),
        pltpu.VMEM((1024,), jnp.int32),
        pltpu.VMEM((1024,), jnp.int32),
        pltpu.VMEM((48,), jnp.int32),
        pltpu.VMEM((48,), jnp.int32),
        pltpu.SemaphoreType.DMA,
        pltpu.SemaphoreType.DMA,
    ),
    compiler_params=_NOLAYOUT)


def _mm_body(x_ref, w_ref, o_ref):
  o_ref[...] = jnp.dot(x_ref[...], w_ref[...],
                       preferred_element_type=jnp.float32)


def _tc_matmul(x, w):
  return pl.pallas_call(
      _mm_body,
      out_shape=jax.ShapeDtypeStruct((x.shape[0], w.shape[1]), jnp.float32),
  )(x, w)


def _mid_body(agg_ref, b_ref, g_ref, be_ref, w_ref, h_ref):
  x = agg_ref[:N, :] + b_ref[...]
  mu = jnp.mean(x, axis=0, keepdims=True)
  var = jnp.mean((x - mu) ** 2, axis=0, keepdims=True)
  x = (x - mu) * lax.rsqrt(var + 1e-5) * g_ref[...] + be_ref[...]
  x = jnp.where(x > 0, x, jnp.exp(x) - 1.0)
  h_ref[...] = jnp.dot(x, w_ref[...], preferred_element_type=jnp.float32)


def _tc_mid(agg, b, gamma, beta, w):
  return pl.pallas_call(
      _mid_body,
      out_shape=jax.ShapeDtypeStruct((N, D), jnp.float32),
  )(agg, b, gamma, beta, w)


def _final_body(agg_ref, b_ref, wl_ref, bl_ref, x_ref, lg_ref):
  x = agg_ref[:N, :] + b_ref[...]
  x_ref[...] = x
  lg_ref[...] = (jnp.dot(x, wl_ref[...], preferred_element_type=jnp.float32)
                 + bl_ref[...])


def _tc_final(agg, b, wl, bl):
  return pl.pallas_call(
      _final_body,
      out_shape=(jax.ShapeDtypeStruct((N, D), jnp.float32),
                 jax.ShapeDtypeStruct((N, L), jnp.float32)),
  )(agg, b, wl, bl)


def kernel(emb, W1, b1, W2, b2, W3, b3, gamma, beta, Wl, bl, edge_index):
  src = edge_index[0].astype(jnp.int32)
  dst = edge_index[1].astype(jnp.int32)
  npad = E_PAD - src.shape[0]
  src_p = jnp.concatenate([src, jnp.zeros((npad,), jnp.int32)])
  dst_p = jnp.concatenate([dst, jnp.full((npad,), -1, jnp.int32)])

  (counts,) = _sc_count(dst_p)
  offs, pr, rounds, starts = _tc_prefix(counts.reshape(NW, 48))
  bsrc, bldst = _sc_bucket(src_p, dst_p, offs.reshape(NW * 32),
                           pr.reshape(NW * 32))
  rounds48 = jnp.pad(rounds, (0, 16))
  starts48 = jnp.pad(starts, (0, 16))

  h1 = _tc_matmul(emb, W1)
  (agg1,) = _sc_agg(bsrc, bldst, h1, rounds48, starts48)
  h2 = _tc_mid(agg1, b1, gamma, beta, W2)
  (agg2,) = _sc_agg(bsrc, bldst, h2, rounds48, starts48)
  h3 = _tc_mid(agg2, b2, gamma, beta, W3)
  (agg3,) = _sc_agg(bsrc, bldst, h3, rounds48, starts48)
  x3, logits = _tc_final(agg3, b3, Wl, bl)
  return (x3, logits)


# speedup vs baseline: 1.4444x; 1.0002x over previous
import functools

import jax
import jax.numpy as jnp
from jax import lax
from jax.experimental import pallas as pl
from jax.experimental.pallas import tpu as pltpu
from jax.experimental.pallas import tpu_sc as plsc

N = 10000
D = 256
E = 160000
L = 40

NC = 2
NS = 16
NW = NC * NS

E_PAD = 163840
ECH = E_PAD // NW
GCH = ECH // 16

OWN = 320
N_PAD = NW * OWN
ACC_ROWS = 328
DUMP = 320
RSIZE = 128
KD = D // 16

RUN_Q = 8
BCAP = 199680
STG = 5376

_MESH = plsc.VectorSubcoreMesh(
    core_axis_name="c", subcore_axis_name="s", num_cores=NC, num_subcores=NS)
_NOLAYOUT = pltpu.CompilerParams(needs_layout_passes=False)


def _sc_count_body(dst_hbm, counts_out, dst_blk, hist):
  c = lax.axis_index("c")
  s = lax.axis_index("s")
  w = c * NS + s

  z16 = jnp.zeros((16,), jnp.float32)
  ones16 = jnp.ones((16,), jnp.float32)
  for i in range(3):
    hist[pl.ds(i * 16, 16)] = z16

  pltpu.sync_copy(dst_hbm.at[pl.ds(w * ECH, ECH)], dst_blk)

  def grp(j, carry):
    dvec = dst_blk[pl.ds(j * 16, 16)]
    ow = ((dvec >> 6) * 3277) >> 14
    ow = jnp.where(dvec >= 0, ow, 32)
    plsc.addupdate_scatter(hist, [ow], ones16)
    return carry
  lax.fori_loop(0, GCH, grp, 0)

  pltpu.sync_copy(hist, counts_out.at[pl.ds(w * 48, 48)])


_sc_count = pl.kernel(
    _sc_count_body,
    out_type=(jax.ShapeDtypeStruct((NW * 48,), jnp.float32),),
    mesh=_MESH,
    scratch_types=(
        pltpu.VMEM((ECH,), jnp.int32),
        pltpu.VMEM((48,), jnp.float32),
    ),
    compiler_params=_NOLAYOUT)


def _prefix_body(cnt_ref, offs_ref, pr_ref, rounds_ref, starts_ref):
  cnt = cnt_ref[...][:, :32]
  pr = jnp.floor((cnt + 7.0) / 8.0) * 8.0
  tot = jnp.sum(pr, axis=0)
  extra = 128.0 * jnp.ceil(tot / 128.0) - tot
  rio = lax.broadcasted_iota(jnp.int32, (32, 32), 0)
  cio = lax.broadcasted_iota(jnp.int32, (32, 32), 1)
  pr = pr + jnp.where(rio == 31, extra[None, :], 0.0)
  tot = tot + extra
  lstrict = (rio > cio).astype(jnp.float32)
  starts = jnp.dot(lstrict, tot[:, None],
                   preferred_element_type=jnp.float32)[:, 0]
  offs = starts[None, :] + jnp.dot(lstrict, pr,
                                   preferred_element_type=jnp.float32)
  offs_ref[...] = offs.astype(jnp.int32)
  pr_ref[...] = pr.astype(jnp.int32)
  rounds_ref[...] = (tot * (1.0 / 128.0)).astype(jnp.int32)
  starts_ref[...] = starts.astype(jnp.int32)


def _tc_prefix(counts2d):
  return pl.pallas_call(
      _prefix_body,
      out_shape=(jax.ShapeDtypeStruct((32, 32), jnp.int32),
                 jax.ShapeDtypeStruct((32, 32), jnp.int32),
                 jax.ShapeDtypeStruct((32,), jnp.int32),
                 jax.ShapeDtypeStruct((32,), jnp.int32)),
  )(counts2d)


def _sc_bucket_body(src_hbm, dst_hbm, offs_hbm, pr_hbm, bsrc_out, bldst_out,
                    src_blk, dst_blk, stage_s, stage_d, offv, prv):
  c = lax.axis_index("c")
  s = lax.axis_index("s")
  w = c * NS + s

  zi16 = jnp.zeros((16,), jnp.int32)
  dump16 = DUMP + (jnp.arange(16, dtype=jnp.int32) & 7)

  pltpu.sync_copy(src_hbm.at[pl.ds(w * ECH, ECH)], src_blk)
  pltpu.sync_copy(dst_hbm.at[pl.ds(w * ECH, ECH)], dst_blk)
  pltpu.sync_copy(offs_hbm, offv)
  pltpu.sync_copy(pr_hbm, prv)

  for o in range(32):
    olo = o * OWN

    def grp(j, cnt):
      dvec = dst_blk[pl.ds(j * 16, 16)]
      svec = src_blk[pl.ds(j * 16, 16)]
      m = (dvec >= olo) & (dvec < olo + OWN)
      plsc.store_compressed(stage_s.at[pl.ds(cnt, 16)], svec, mask=m)
      plsc.store_compressed(stage_d.at[pl.ds(cnt, 16)], dvec - olo, mask=m)
      return cnt + jnp.sum(m.astype(jnp.int32))
    cnt = lax.fori_loop(0, GCH, grp, 0)

    lofs = offv[pl.ds(w * 32 + (o // 16) * 16, 16)]
    lpr = prv[pl.ds(w * 32 + (o // 16) * 16, 16)]
    off_o = pl.multiple_of(lofs[o % 16], 8)
    pr_o = lpr[o % 16]

    npg = (pr_o - cnt + 15) // 16

    def padg(i, carry):
      stage_s[pl.ds(cnt + i * 16, 16)] = zi16
      stage_d[pl.ds(cnt + i * 16, 16)] = dump16
      return carry
    lax.fori_loop(0, npg, padg, 0)

    nch = pr_o // RUN_Q

    def dmac(i, carry):
      pltpu.sync_copy(stage_s.at[pl.ds(i * RUN_Q, RUN_Q)],
                      bsrc_out.at[pl.ds(off_o + i * RUN_Q, RUN_Q)])
      pltpu.sync_copy(stage_d.at[pl.ds(i * RUN_Q, RUN_Q)],
                      bldst_out.at[pl.ds(off_o + i * RUN_Q, RUN_Q)])
      return carry
    lax.fori_loop(0, nch, dmac, 0)


_sc_bucket = pl.kernel(
    _sc_bucket_body,
    out_type=(jax.ShapeDtypeStruct((BCAP,), jnp.int32),
              jax.ShapeDtypeStruct((BCAP,), jnp.int32)),
    mesh=_MESH,
    scratch_types=(
        pltpu.VMEM((ECH,), jnp.int32),
        pltpu.VMEM((ECH,), jnp.int32),
        pltpu.VMEM((STG,), jnp.int32),
        pltpu.VMEM((STG,), jnp.int32),
        pltpu.VMEM((NW * 32,), jnp.int32),
        pltpu.VMEM((NW * 32,), jnp.int32),
    ),
    compiler_params=_NOLAYOUT)


def _sc_agg_body(bsrc, bldst, h_hbm, rounds_hbm, starts_hbm, agg_out,
                 gidx0, gidx1, sdx0, sdx1, rows0, rows1, acc, deg,
                 bsv, blv, rv, sv, sem0, sem1):
  c = lax.axis_index("c")
  s = lax.axis_index("s")
  w = c * NS + s
  wlo = w * OWN

  z16 = jnp.zeros((16,), jnp.float32)
  ones16 = jnp.ones((16,), jnp.float32)

  def zacc(i, carry):
    for k in range(KD):
      acc[i, pl.ds(k * 16, 16)] = z16
    return carry
  lax.fori_loop(0, ACC_ROWS, zacc, 0)
  def zdeg(i, carry):
    deg[pl.ds(i * 16, 16)] = z16
    return carry
  lax.fori_loop(0, ACC_ROWS // 8, zdeg, 0)

  pltpu.sync_copy(rounds_hbm, rv)
  pltpu.sync_copy(starts_hbm, sv)
  lanes = jnp.arange(16, dtype=jnp.int32)
  msk = lanes == s
  zi = jnp.zeros((16,), jnp.int32)
  nr = jnp.sum(jnp.where(msk, rv[pl.ds(c * 16, 16)], zi))
  st = pl.multiple_of(jnp.sum(jnp.where(msk, sv[pl.ds(c * 16, 16)], zi)),
                      RSIZE)

  def accumulate(sdx_b, rows_b):
    for j in range(4):
      plsc.addupdate_scatter(deg, [sdx_b[pl.ds(j * 16, 16)]], ones16)

    def acc_grp(i16, carry2):
      lvec = sdx_b[pl.ds(i16 * 16, 16)]
      base = i16 * 16
      for lane in range(16):
        r_own = lvec[lane]
        for k in range(KD):
          acc[r_own, pl.ds(k * 16, 16)] = (
              acc[r_own, pl.ds(k * 16, 16)]
              + rows_b[base + lane, pl.ds(k * 16, 16)])
      return carry2
    lax.fori_loop(0, 4, acc_grp, 0)

  def stage(gidx_b, sdx_b, off):
    for q in range(4):
      gidx_b[pl.ds(q * 16, 16)] = bsv[pl.ds(off + q * 16, 16)]
      sdx_b[pl.ds(q * 16, 16)] = blv[pl.ds(off + q * 16, 16)]

  def pair(p, carry):
    @pl.when((p & 7) == 0)
    def _():
      pltpu.sync_copy(bsrc.at[pl.ds(st + p * 128, 1024)], bsv)
      pltpu.sync_copy(bldst.at[pl.ds(st + p * 128, 1024)], blv)

    off0 = (p & 7) * 128
    stage(gidx0, sdx0, off0)
    cp0 = pltpu.async_copy(h_hbm.at[gidx0], rows0, sem0)

    @pl.when(p > 0)
    def _():
      pltpu.make_async_copy(h_hbm.at[gidx1], rows1, sem1).wait()
      accumulate(sdx1, rows1)

    stage(gidx1, sdx1, off0 + 64)
    pltpu.async_copy(h_hbm.at[gidx1], rows1, sem1)

    cp0.wait()
    accumulate(sdx0, rows0)
    return carry
  lax.fori_loop(0, nr, pair, 0)

  @pl.when(nr > 0)
  def _():
    pltpu.make_async_copy(h_hbm.at[gidx1], rows1, sem1).wait()
    accumulate(sdx1, rows1)

  def norm(i16, carry):
    dv = jnp.maximum(deg[pl.ds(i16 * 16, 16)], 1.0)
    rinv = ones16 / dv
    for lane in range(16):
      r_own = i16 * 16 + lane
      dsp = rinv[lane] * ones16
      for k in range(KD):
        acc[r_own, pl.ds(k * 16, 16)] = acc[r_own, pl.ds(k * 16, 16)] * dsp
    return carry
  lax.fori_loop(0, OWN // 16, norm, 0)

  pltpu.sync_copy(acc.at[pl.ds(0, OWN)], agg_out.at[pl.ds(wlo, OWN)])


_sc_agg = pl.kernel(
    _sc_agg_body,
    out_type=(jax.ShapeDtypeStruct((N_PAD, D), jnp.float32),),
    mesh=_MESH,
    scratch_types=(
        pltpu.VMEM((64,), jnp.int32),
        pltpu.VMEM((64,), jnp.int32),
        pltpu.VMEM((64,), jnp.int32),
        pltpu.VMEM((64,), jnp.int32),
        pltpu.VMEM((64, D), jnp.float32),
        pltpu.VMEM((64, D), jnp.float32),
        pltpu.VMEM((ACC_ROWS, D), jnp.float32),
        pltpu.VMEM((ACC_ROWS,), jnp.float32),
        pltpu.VMEM((1024,), jnp.int32),
        pltpu.VMEM((1024,), jnp.int32),
        pltpu.VMEM((48,), jnp.int32),
        pltpu.VMEM((48,), jnp.int32),
        pltpu.SemaphoreType.DMA,
        pltpu.SemaphoreType.DMA,
    ),
    compiler_params=_NOLAYOUT)


def _mm_body(x_ref, w_ref, o_ref):
  o_ref[...] = jnp.dot(x_ref[...], w_ref[...],
                       preferred_element_type=jnp.float32)


def _tc_matmul(x, w):
  return pl.pallas_call(
      _mm_body,
      out_shape=jax.ShapeDtypeStruct((x.shape[0], w.shape[1]), jnp.float32),
  )(x, w)


def _mid_body(agg_ref, b_ref, g_ref, be_ref, w_ref, h_ref):
  x = agg_ref[:N, :] + b_ref[...]
  mu = jnp.mean(x, axis=0, keepdims=True)
  var = jnp.mean((x - mu) ** 2, axis=0, keepdims=True)
  x = (x - mu) * lax.rsqrt(var + 1e-5) * g_ref[...] + be_ref[...]
  x = jnp.where(x > 0, x, jnp.exp(x) - 1.0)
  h_ref[...] = jnp.dot(x, w_ref[...], preferred_element_type=jnp.float32)


def _tc_mid(agg, b, gamma, beta, w):
  return pl.pallas_call(
      _mid_body,
      out_shape=jax.ShapeDtypeStruct((N, D), jnp.float32),
  )(agg, b, gamma, beta, w)


def _final_body(agg_ref, b_ref, wl_ref, bl_ref, x_ref, lg_ref):
  x = agg_ref[:N, :] + b_ref[...]
  x_ref[...] = x
  lg_ref[...] = (jnp.dot(x, wl_ref[...], preferred_element_type=jnp.float32)
                 + bl_ref[...])


def _tc_final(agg, b, wl, bl):
  return pl.pallas_call(
      _final_body,
      out_shape=(jax.ShapeDtypeStruct((N, D), jnp.float32),
                 jax.ShapeDtypeStruct((N, L), jnp.float32)),
  )(agg, b, wl, bl)


def kernel(emb, W1, b1, W2, b2, W3, b3, gamma, beta, Wl, bl, edge_index):
  src = edge_index[0].astype(jnp.int32)
  dst = edge_index[1].astype(jnp.int32)
  npad = E_PAD - src.shape[0]
  src_p = jnp.concatenate([src, jnp.zeros((npad,), jnp.int32)])
  dst_p = jnp.concatenate([dst, jnp.full((npad,), -1, jnp.int32)])

  (counts,) = _sc_count(dst_p)
  offs, pr, rounds, starts = _tc_prefix(counts.reshape(NW, 48))
  bsrc, bldst = _sc_bucket(src_p, dst_p, offs.reshape(NW * 32),
                           pr.reshape(NW * 32))
  rounds48 = jnp.pad(rounds, (0, 16))
  starts48 = jnp.pad(starts, (0, 16))

  h1 = _tc_matmul(emb, W1)
  (agg1,) = _sc_agg(bsrc, bldst, h1, rounds48, starts48)
  h2 = _tc_mid(agg1, b1, gamma, beta, W2)
  (agg2,) = _sc_agg(bsrc, bldst, h2, rounds48, starts48)
  h3 = _tc_mid(agg2, b2, gamma, beta, W3)
  (agg3,) = _sc_agg(bsrc, bldst, h3, rounds48, starts48)
  x3, logits = _tc_final(agg3, b3, Wl, bl)
  return (x3, logits)


# speedup vs baseline: 1.7653x; 1.2222x over previous
import functools

import jax
import jax.numpy as jnp
from jax import lax
from jax.experimental import pallas as pl
from jax.experimental.pallas import tpu as pltpu
from jax.experimental.pallas import tpu_sc as plsc

N = 10000
D = 256
E = 160000
L = 40

NC = 2
NS = 16
NW = NC * NS

E_PAD = 163840
ECH = E_PAD // NW
GCH = ECH // 16

OWN = 320
N_PAD = NW * OWN
ACC_ROWS = 328
DUMP = 320
RSIZE = 128
KD = D // 16

RUN_Q = 8
BCAP = 199680
STG = 5376

_MESH = plsc.VectorSubcoreMesh(
    core_axis_name="c", subcore_axis_name="s", num_cores=NC, num_subcores=NS)
_NOLAYOUT = pltpu.CompilerParams(needs_layout_passes=False)


def _sc_count_body(dst_hbm, counts_out, dst_blk, hist):
  c = lax.axis_index("c")
  s = lax.axis_index("s")
  w = c * NS + s

  z16 = jnp.zeros((16,), jnp.float32)
  ones16 = jnp.ones((16,), jnp.float32)
  for i in range(3):
    hist[pl.ds(i * 16, 16)] = z16

  pltpu.sync_copy(dst_hbm.at[pl.ds(w * ECH, ECH)], dst_blk)

  def grp(j, carry):
    dvec = dst_blk[pl.ds(j * 16, 16)]
    ow = ((dvec >> 6) * 3277) >> 14
    ow = jnp.where(dvec >= 0, ow, 32)
    plsc.addupdate_scatter(hist, [ow], ones16)
    return carry
  lax.fori_loop(0, GCH, grp, 0)

  pltpu.sync_copy(hist, counts_out.at[pl.ds(w * 48, 48)])


_sc_count = pl.kernel(
    _sc_count_body,
    out_type=(jax.ShapeDtypeStruct((NW * 48,), jnp.float32),),
    mesh=_MESH,
    scratch_types=(
        pltpu.VMEM((ECH,), jnp.int32),
        pltpu.VMEM((48,), jnp.float32),
    ),
    compiler_params=_NOLAYOUT)


def _prefix_body(cnt_ref, offs_ref, pr_ref, rounds_ref, starts_ref):
  cnt = cnt_ref[...][:, :32]
  pr = jnp.floor((cnt + 7.0) / 8.0) * 8.0
  tot = jnp.sum(pr, axis=0)
  extra = 128.0 * jnp.ceil(tot / 128.0) - tot
  rio = lax.broadcasted_iota(jnp.int32, (32, 32), 0)
  cio = lax.broadcasted_iota(jnp.int32, (32, 32), 1)
  pr = pr + jnp.where(rio == 31, extra[None, :], 0.0)
  tot = tot + extra
  lstrict = (rio > cio).astype(jnp.float32)
  starts = jnp.dot(lstrict, tot[:, None],
                   preferred_element_type=jnp.float32)[:, 0]
  offs = starts[None, :] + jnp.dot(lstrict, pr,
                                   preferred_element_type=jnp.float32)
  offs_ref[...] = offs.astype(jnp.int32)
  pr_ref[...] = pr.astype(jnp.int32)
  rounds_ref[...] = (tot * (1.0 / 128.0)).astype(jnp.int32)
  starts_ref[...] = starts.astype(jnp.int32)


def _tc_prefix(counts2d):
  return pl.pallas_call(
      _prefix_body,
      out_shape=(jax.ShapeDtypeStruct((32, 32), jnp.int32),
                 jax.ShapeDtypeStruct((32, 32), jnp.int32),
                 jax.ShapeDtypeStruct((32,), jnp.int32),
                 jax.ShapeDtypeStruct((32,), jnp.int32)),
  )(counts2d)


def _sc_bucket_body(src_hbm, dst_hbm, offs_hbm, pr_hbm, bsrc_out, bldst_out,
                    src_blk, dst_blk, stage_s, stage_d, offv, prv):
  c = lax.axis_index("c")
  s = lax.axis_index("s")
  w = c * NS + s

  zi16 = jnp.zeros((16,), jnp.int32)
  dump16 = DUMP + (jnp.arange(16, dtype=jnp.int32) & 7)

  pltpu.sync_copy(src_hbm.at[pl.ds(w * ECH, ECH)], src_blk)
  pltpu.sync_copy(dst_hbm.at[pl.ds(w * ECH, ECH)], dst_blk)
  pltpu.sync_copy(offs_hbm, offv)
  pltpu.sync_copy(pr_hbm, prv)

  for o in range(32):
    olo = o * OWN

    def grp(j, cnt):
      dvec = dst_blk[pl.ds(j * 16, 16)]
      svec = src_blk[pl.ds(j * 16, 16)]
      m = (dvec >= olo) & (dvec < olo + OWN)
      plsc.store_compressed(stage_s.at[pl.ds(cnt, 16)], svec, mask=m)
      plsc.store_compressed(stage_d.at[pl.ds(cnt, 16)], dvec - olo, mask=m)
      return cnt + jnp.sum(m.astype(jnp.int32))
    cnt = lax.fori_loop(0, GCH, grp, 0)

    lofs = offv[pl.ds(w * 32 + (o // 16) * 16, 16)]
    lpr = prv[pl.ds(w * 32 + (o // 16) * 16, 16)]
    off_o = pl.multiple_of(lofs[o % 16], 8)
    pr_o = lpr[o % 16]

    npg = (pr_o - cnt + 15) // 16

    def padg(i, carry):
      stage_s[pl.ds(cnt + i * 16, 16)] = zi16
      stage_d[pl.ds(cnt + i * 16, 16)] = dump16
      return carry
    lax.fori_loop(0, npg, padg, 0)

    nch = pr_o // RUN_Q

    def dmac(i, carry):
      pltpu.sync_copy(stage_s.at[pl.ds(i * RUN_Q, RUN_Q)],
                      bsrc_out.at[pl.ds(off_o + i * RUN_Q, RUN_Q)])
      pltpu.sync_copy(stage_d.at[pl.ds(i * RUN_Q, RUN_Q)],
                      bldst_out.at[pl.ds(off_o + i * RUN_Q, RUN_Q)])
      return carry
    lax.fori_loop(0, nch, dmac, 0)


_sc_bucket = pl.kernel(
    _sc_bucket_body,
    out_type=(jax.ShapeDtypeStruct((BCAP,), jnp.int32),
              jax.ShapeDtypeStruct((BCAP,), jnp.int32)),
    mesh=_MESH,
    scratch_types=(
        pltpu.VMEM((ECH,), jnp.int32),
        pltpu.VMEM((ECH,), jnp.int32),
        pltpu.VMEM((STG,), jnp.int32),
        pltpu.VMEM((STG,), jnp.int32),
        pltpu.VMEM((NW * 32,), jnp.int32),
        pltpu.VMEM((NW * 32,), jnp.int32),
    ),
    compiler_params=_NOLAYOUT)


def _sc_agg_body(bsrc, bldst, h_hbm, rounds_hbm, starts_hbm, agg_out,
                 gidx0, gidx1, sdx0, sdx1, rows0, rows1, acc, deg,
                 bsv, blv, rv, sv, sem0, sem1):
  c = lax.axis_index("c")
  s = lax.axis_index("s")
  w = c * NS + s
  wlo = w * OWN

  z16 = jnp.zeros((16,), jnp.float32)
  ones16 = jnp.ones((16,), jnp.float32)

  def zacc(i, carry):
    for k in range(KD):
      acc[i, pl.ds(k * 16, 16)] = z16
    return carry
  lax.fori_loop(0, ACC_ROWS, zacc, 0)
  def zdeg(i, carry):
    deg[pl.ds(i * 16, 16)] = z16
    return carry
  lax.fori_loop(0, ACC_ROWS // 8, zdeg, 0)

  pltpu.sync_copy(rounds_hbm, rv)
  pltpu.sync_copy(starts_hbm, sv)
  lanes = jnp.arange(16, dtype=jnp.int32)
  msk = lanes == s
  zi = jnp.zeros((16,), jnp.int32)
  nr = jnp.sum(jnp.where(msk, rv[pl.ds(c * 16, 16)], zi))
  st = pl.multiple_of(jnp.sum(jnp.where(msk, sv[pl.ds(c * 16, 16)], zi)),
                      RSIZE)

  def accumulate(sdx_b, rows_b):
    for j in range(4):
      plsc.addupdate_scatter(deg, [sdx_b[pl.ds(j * 16, 16)]], ones16)

    def acc_grp(i16, carry2):
      lvec = sdx_b[pl.ds(i16 * 16, 16)]
      base = i16 * 16
      for lane in range(16):
        r_own = lvec[lane]
        for k in range(KD):
          plsc.addupdate(acc.at[r_own, pl.ds(k * 16, 16)],
                         rows_b[base + lane, pl.ds(k * 16, 16)])
      return carry2
    lax.fori_loop(0, 4, acc_grp, 0)

  def stage(gidx_b, sdx_b, off):
    for q in range(4):
      gidx_b[pl.ds(q * 16, 16)] = bsv[pl.ds(off + q * 16, 16)]
      sdx_b[pl.ds(q * 16, 16)] = blv[pl.ds(off + q * 16, 16)]

  def pair(p, carry):
    @pl.when((p & 7) == 0)
    def _():
      pltpu.sync_copy(bsrc.at[pl.ds(st + p * 128, 1024)], bsv)
      pltpu.sync_copy(bldst.at[pl.ds(st + p * 128, 1024)], blv)

    off0 = (p & 7) * 128
    stage(gidx0, sdx0, off0)
    cp0 = pltpu.async_copy(h_hbm.at[gidx0], rows0, sem0)

    @pl.when(p > 0)
    def _():
      pltpu.make_async_copy(h_hbm.at[gidx1], rows1, sem1).wait()
      accumulate(sdx1, rows1)

    stage(gidx1, sdx1, off0 + 64)
    pltpu.async_copy(h_hbm.at[gidx1], rows1, sem1)

    cp0.wait()
    accumulate(sdx0, rows0)
    return carry
  lax.fori_loop(0, nr, pair, 0)

  @pl.when(nr > 0)
  def _():
    pltpu.make_async_copy(h_hbm.at[gidx1], rows1, sem1).wait()
    accumulate(sdx1, rows1)

  def norm(i16, carry):
    dv = jnp.maximum(deg[pl.ds(i16 * 16, 16)], 1.0)
    rinv = ones16 / dv
    for lane in range(16):
      r_own = i16 * 16 + lane
      dsp = rinv[lane] * ones16
      for k in range(KD):
        acc[r_own, pl.ds(k * 16, 16)] = acc[r_own, pl.ds(k * 16, 16)] * dsp
    return carry
  lax.fori_loop(0, OWN // 16, norm, 0)

  pltpu.sync_copy(acc.at[pl.ds(0, OWN)], agg_out.at[pl.ds(wlo, OWN)])


_sc_agg = pl.kernel(
    _sc_agg_body,
    out_type=(jax.ShapeDtypeStruct((N_PAD, D), jnp.float32),),
    mesh=_MESH,
    scratch_types=(
        pltpu.VMEM((64,), jnp.int32),
        pltpu.VMEM((64,), jnp.int32),
        pltpu.VMEM((64,), jnp.int32),
        pltpu.VMEM((64,), jnp.int32),
        pltpu.VMEM((64, D), jnp.float32),
        pltpu.VMEM((64, D), jnp.float32),
        pltpu.VMEM((ACC_ROWS, D), jnp.float32),
        pltpu.VMEM((ACC_ROWS,), jnp.float32),
        pltpu.VMEM((1024,), jnp.int32),
        pltpu.VMEM((1024,), jnp.int32),
        pltpu.VMEM((48,), jnp.int32),
        pltpu.VMEM((48,), jnp.int32),
        pltpu.SemaphoreType.DMA,
        pltpu.SemaphoreType.DMA,
    ),
    compiler_params=_NOLAYOUT)


def _mm_body(x_ref, w_ref, o_ref):
  o_ref[...] = jnp.dot(x_ref[...], w_ref[...],
                       preferred_element_type=jnp.float32)


def _tc_matmul(x, w):
  return pl.pallas_call(
      _mm_body,
      out_shape=jax.ShapeDtypeStruct((x.shape[0], w.shape[1]), jnp.float32),
  )(x, w)


def _mid_body(agg_ref, b_ref, g_ref, be_ref, w_ref, h_ref):
  x = agg_ref[:N, :] + b_ref[...]
  mu = jnp.mean(x, axis=0, keepdims=True)
  var = jnp.mean((x - mu) ** 2, axis=0, keepdims=True)
  x = (x - mu) * lax.rsqrt(var + 1e-5) * g_ref[...] + be_ref[...]
  x = jnp.where(x > 0, x, jnp.exp(x) - 1.0)
  h_ref[...] = jnp.dot(x, w_ref[...], preferred_element_type=jnp.float32)


def _tc_mid(agg, b, gamma, beta, w):
  return pl.pallas_call(
      _mid_body,
      out_shape=jax.ShapeDtypeStruct((N, D), jnp.float32),
  )(agg, b, gamma, beta, w)


def _final_body(agg_ref, b_ref, wl_ref, bl_ref, x_ref, lg_ref):
  x = agg_ref[:N, :] + b_ref[...]
  x_ref[...] = x
  lg_ref[...] = (jnp.dot(x, wl_ref[...], preferred_element_type=jnp.float32)
                 + bl_ref[...])


def _tc_final(agg, b, wl, bl):
  return pl.pallas_call(
      _final_body,
      out_shape=(jax.ShapeDtypeStruct((N, D), jnp.float32),
                 jax.ShapeDtypeStruct((N, L), jnp.float32)),
  )(agg, b, wl, bl)


def kernel(emb, W1, b1, W2, b2, W3, b3, gamma, beta, Wl, bl, edge_index):
  src = edge_index[0].astype(jnp.int32)
  dst = edge_index[1].astype(jnp.int32)
  npad = E_PAD - src.shape[0]
  src_p = jnp.concatenate([src, jnp.zeros((npad,), jnp.int32)])
  dst_p = jnp.concatenate([dst, jnp.full((npad,), -1, jnp.int32)])

  (counts,) = _sc_count(dst_p)
  offs, pr, rounds, starts = _tc_prefix(counts.reshape(NW, 48))
  bsrc, bldst = _sc_bucket(src_p, dst_p, offs.reshape(NW * 32),
                           pr.reshape(NW * 32))
  rounds48 = jnp.pad(rounds, (0, 16))
  starts48 = jnp.pad(starts, (0, 16))

  h1 = _tc_matmul(emb, W1)
  (agg1,) = _sc_agg(bsrc, bldst, h1, rounds48, starts48)
  h2 = _tc_mid(agg1, b1, gamma, beta, W2)
  (agg2,) = _sc_agg(bsrc, bldst, h2, rounds48, starts48)
  h3 = _tc_mid(agg2, b2, gamma, beta, W3)
  (agg3,) = _sc_agg(bsrc, bldst, h3, rounds48, starts48)
  x3, logits = _tc_final(agg3, b3, Wl, bl)
  return (x3, logits)


# speedup vs baseline: 1.8865x; 1.0687x over previous
import functools

import jax
import jax.numpy as jnp
from jax import lax
from jax.experimental import pallas as pl
from jax.experimental.pallas import tpu as pltpu
from jax.experimental.pallas import tpu_sc as plsc

N = 10000
D = 256
E = 160000
L = 40

NC = 2
NS = 16
NW = NC * NS

E_PAD = 163840
ECH = E_PAD // NW
GCH = ECH // 16

OWN = 320
N_PAD = NW * OWN
ACC_ROWS = 328
DUMP = 320
RSIZE = 128
KD = D // 16

RUN_Q = 8
BCAP = 199680
STG = 5376

_MESH = plsc.VectorSubcoreMesh(
    core_axis_name="c", subcore_axis_name="s", num_cores=NC, num_subcores=NS)
_NOLAYOUT = pltpu.CompilerParams(needs_layout_passes=False)


def _sc_count_body(dst_hbm, counts_out, dst_blk, hist):
  c = lax.axis_index("c")
  s = lax.axis_index("s")
  w = c * NS + s

  z16 = jnp.zeros((16,), jnp.float32)
  ones16 = jnp.ones((16,), jnp.float32)
  for i in range(3):
    hist[pl.ds(i * 16, 16)] = z16

  pltpu.sync_copy(dst_hbm.at[pl.ds(w * ECH, ECH)], dst_blk)

  def grp(j, carry):
    dvec = dst_blk[pl.ds(j * 16, 16)]
    ow = ((dvec >> 6) * 3277) >> 14
    ow = jnp.where(dvec >= 0, ow, 32)
    plsc.addupdate_scatter(hist, [ow], ones16)
    return carry
  lax.fori_loop(0, GCH, grp, 0)

  pltpu.sync_copy(hist, counts_out.at[pl.ds(w * 48, 48)])


_sc_count = pl.kernel(
    _sc_count_body,
    out_type=(jax.ShapeDtypeStruct((NW * 48,), jnp.float32),),
    mesh=_MESH,
    scratch_types=(
        pltpu.VMEM((ECH,), jnp.int32),
        pltpu.VMEM((48,), jnp.float32),
    ),
    compiler_params=_NOLAYOUT)


def _prefix_body(cnt_ref, offs_ref, pr_ref, rounds_ref, starts_ref):
  cnt = cnt_ref[...][:, :32]
  pr = jnp.floor((cnt + 7.0) / 8.0) * 8.0
  tot = jnp.sum(pr, axis=0)
  extra = 128.0 * jnp.ceil(tot / 128.0) - tot
  rio = lax.broadcasted_iota(jnp.int32, (32, 32), 0)
  cio = lax.broadcasted_iota(jnp.int32, (32, 32), 1)
  pr = pr + jnp.where(rio == 31, extra[None, :], 0.0)
  tot = tot + extra
  lstrict = (rio > cio).astype(jnp.float32)
  starts = jnp.dot(lstrict, tot[:, None],
                   preferred_element_type=jnp.float32)[:, 0]
  offs = starts[None, :] + jnp.dot(lstrict, pr,
                                   preferred_element_type=jnp.float32)
  offs_ref[...] = offs.astype(jnp.int32)
  pr_ref[...] = pr.astype(jnp.int32)
  rounds_ref[...] = (tot * (1.0 / 128.0)).astype(jnp.int32)
  starts_ref[...] = starts.astype(jnp.int32)


def _tc_prefix(counts2d):
  return pl.pallas_call(
      _prefix_body,
      out_shape=(jax.ShapeDtypeStruct((32, 32), jnp.int32),
                 jax.ShapeDtypeStruct((32, 32), jnp.int32),
                 jax.ShapeDtypeStruct((32,), jnp.int32),
                 jax.ShapeDtypeStruct((32,), jnp.int32)),
  )(counts2d)


def _sc_bucket_body(src_hbm, dst_hbm, offs_hbm, pr_hbm, bsrc_out, bldst_out,
                    src_blk, dst_blk, stage_s, stage_d, offv, prv):
  c = lax.axis_index("c")
  s = lax.axis_index("s")
  w = c * NS + s

  zi16 = jnp.zeros((16,), jnp.int32)
  dump16 = DUMP + (jnp.arange(16, dtype=jnp.int32) & 7)

  pltpu.sync_copy(src_hbm.at[pl.ds(w * ECH, ECH)], src_blk)
  pltpu.sync_copy(dst_hbm.at[pl.ds(w * ECH, ECH)], dst_blk)
  pltpu.sync_copy(offs_hbm, offv)
  pltpu.sync_copy(pr_hbm, prv)

  for o in range(32):
    olo = o * OWN

    def grp(j, cnt):
      dvec = dst_blk[pl.ds(j * 16, 16)]
      svec = src_blk[pl.ds(j * 16, 16)]
      m = (dvec >= olo) & (dvec < olo + OWN)
      plsc.store_compressed(stage_s.at[pl.ds(cnt, 16)], svec, mask=m)
      plsc.store_compressed(stage_d.at[pl.ds(cnt, 16)], dvec - olo, mask=m)
      return cnt + jnp.sum(m.astype(jnp.int32))
    cnt = lax.fori_loop(0, GCH, grp, 0)

    lofs = offv[pl.ds(w * 32 + (o // 16) * 16, 16)]
    lpr = prv[pl.ds(w * 32 + (o // 16) * 16, 16)]
    off_o = pl.multiple_of(lofs[o % 16], 8)
    pr_o = lpr[o % 16]

    npg = (pr_o - cnt + 15) // 16

    def padg(i, carry):
      stage_s[pl.ds(cnt + i * 16, 16)] = zi16
      stage_d[pl.ds(cnt + i * 16, 16)] = dump16
      return carry
    lax.fori_loop(0, npg, padg, 0)

    nch = pr_o // RUN_Q

    def dmac(i, carry):
      pltpu.sync_copy(stage_s.at[pl.ds(i * RUN_Q, RUN_Q)],
                      bsrc_out.at[pl.ds(off_o + i * RUN_Q, RUN_Q)])
      pltpu.sync_copy(stage_d.at[pl.ds(i * RUN_Q, RUN_Q)],
                      bldst_out.at[pl.ds(off_o + i * RUN_Q, RUN_Q)])
      return carry
    lax.fori_loop(0, nch, dmac, 0)


_sc_bucket = pl.kernel(
    _sc_bucket_body,
    out_type=(jax.ShapeDtypeStruct((BCAP,), jnp.int32),
              jax.ShapeDtypeStruct((BCAP,), jnp.int32)),
    mesh=_MESH,
    scratch_types=(
        pltpu.VMEM((ECH,), jnp.int32),
        pltpu.VMEM((ECH,), jnp.int32),
        pltpu.VMEM((STG,), jnp.int32),
        pltpu.VMEM((STG,), jnp.int32),
        pltpu.VMEM((NW * 32,), jnp.int32),
        pltpu.VMEM((NW * 32,), jnp.int32),
    ),
    compiler_params=_NOLAYOUT)


def _sc_agg_body(bsrc, bldst, h_hbm, rounds_hbm, starts_hbm, agg_out,
                 gidx0, gidx1, sdx0, sdx1, rows0, rows1, acc, deg,
                 bsv, blv, rv, sv, sem0, sem1):
  c = lax.axis_index("c")
  s = lax.axis_index("s")
  w = c * NS + s
  wlo = w * OWN

  z16 = jnp.zeros((16,), jnp.float32)
  ones16 = jnp.ones((16,), jnp.float32)

  def zacc(i, carry):
    for k in range(KD):
      acc[i, pl.ds(k * 16, 16)] = z16
    return carry
  lax.fori_loop(0, ACC_ROWS, zacc, 0)
  def zdeg(i, carry):
    deg[pl.ds(i * 16, 16)] = z16
    return carry
  lax.fori_loop(0, ACC_ROWS // 8, zdeg, 0)

  pltpu.sync_copy(rounds_hbm, rv)
  pltpu.sync_copy(starts_hbm, sv)
  lanes = jnp.arange(16, dtype=jnp.int32)
  msk = lanes == s
  zi = jnp.zeros((16,), jnp.int32)
  nr = jnp.sum(jnp.where(msk, rv[pl.ds(c * 16, 16)], zi))
  st = pl.multiple_of(jnp.sum(jnp.where(msk, sv[pl.ds(c * 16, 16)], zi)),
                      RSIZE)

  def accumulate(sdx_b, rows_b):
    for j in range(4):
      plsc.addupdate_scatter(deg, [sdx_b[pl.ds(j * 16, 16)]], ones16)

    @plsc.parallel_loop(0, 4)
    def acc_grp(i16):
      lvec = sdx_b[pl.ds(i16 * 16, 16)]
      base = i16 * 16
      for lane in range(16):
        r_own = lvec[lane]
        for k in range(KD):
          plsc.addupdate(acc.at[r_own, pl.ds(k * 16, 16)],
                         rows_b[base + lane, pl.ds(k * 16, 16)])

  def stage(gidx_b, sdx_b, off):
    for q in range(4):
      gidx_b[pl.ds(q * 16, 16)] = bsv[pl.ds(off + q * 16, 16)]
      sdx_b[pl.ds(q * 16, 16)] = blv[pl.ds(off + q * 16, 16)]

  def pair(p, carry):
    @pl.when((p & 7) == 0)
    def _():
      pltpu.sync_copy(bsrc.at[pl.ds(st + p * 128, 1024)], bsv)
      pltpu.sync_copy(bldst.at[pl.ds(st + p * 128, 1024)], blv)

    off0 = (p & 7) * 128
    stage(gidx0, sdx0, off0)
    cp0 = pltpu.async_copy(h_hbm.at[gidx0], rows0, sem0)

    @pl.when(p > 0)
    def _():
      pltpu.make_async_copy(h_hbm.at[gidx1], rows1, sem1).wait()
      accumulate(sdx1, rows1)

    stage(gidx1, sdx1, off0 + 64)
    pltpu.async_copy(h_hbm.at[gidx1], rows1, sem1)

    cp0.wait()
    accumulate(sdx0, rows0)
    return carry
  lax.fori_loop(0, nr, pair, 0)

  @pl.when(nr > 0)
  def _():
    pltpu.make_async_copy(h_hbm.at[gidx1], rows1, sem1).wait()
    accumulate(sdx1, rows1)

  def norm(i16, carry):
    dv = jnp.maximum(deg[pl.ds(i16 * 16, 16)], 1.0)
    rinv = ones16 / dv
    for lane in range(16):
      r_own = i16 * 16 + lane
      dsp = rinv[lane] * ones16
      for k in range(KD):
        acc[r_own, pl.ds(k * 16, 16)] = acc[r_own, pl.ds(k * 16, 16)] * dsp
    return carry
  lax.fori_loop(0, OWN // 16, norm, 0)

  pltpu.sync_copy(acc.at[pl.ds(0, OWN)], agg_out.at[pl.ds(wlo, OWN)])


_sc_agg = pl.kernel(
    _sc_agg_body,
    out_type=(jax.ShapeDtypeStruct((N_PAD, D), jnp.float32),),
    mesh=_MESH,
    scratch_types=(
        pltpu.VMEM((64,), jnp.int32),
        pltpu.VMEM((64,), jnp.int32),
        pltpu.VMEM((64,), jnp.int32),
        pltpu.VMEM((64,), jnp.int32),
        pltpu.VMEM((64, D), jnp.float32),
        pltpu.VMEM((64, D), jnp.float32),
        pltpu.VMEM((ACC_ROWS, D), jnp.float32),
        pltpu.VMEM((ACC_ROWS,), jnp.float32),
        pltpu.VMEM((1024,), jnp.int32),
        pltpu.VMEM((1024,), jnp.int32),
        pltpu.VMEM((48,), jnp.int32),
        pltpu.VMEM((48,), jnp.int32),
        pltpu.SemaphoreType.DMA,
        pltpu.SemaphoreType.DMA,
    ),
    compiler_params=_NOLAYOUT)


def _mm_body(x_ref, w_ref, o_ref):
  o_ref[...] = jnp.dot(x_ref[...], w_ref[...],
                       preferred_element_type=jnp.float32)


def _tc_matmul(x, w):
  return pl.pallas_call(
      _mm_body,
      out_shape=jax.ShapeDtypeStruct((x.shape[0], w.shape[1]), jnp.float32),
  )(x, w)


def _mid_body(agg_ref, b_ref, g_ref, be_ref, w_ref, h_ref):
  x = agg_ref[:N, :] + b_ref[...]
  mu = jnp.mean(x, axis=0, keepdims=True)
  var = jnp.mean((x - mu) ** 2, axis=0, keepdims=True)
  x = (x - mu) * lax.rsqrt(var + 1e-5) * g_ref[...] + be_ref[...]
  x = jnp.where(x > 0, x, jnp.exp(x) - 1.0)
  h_ref[...] = jnp.dot(x, w_ref[...], preferred_element_type=jnp.float32)


def _tc_mid(agg, b, gamma, beta, w):
  return pl.pallas_call(
      _mid_body,
      out_shape=jax.ShapeDtypeStruct((N, D), jnp.float32),
  )(agg, b, gamma, beta, w)


def _final_body(agg_ref, b_ref, wl_ref, bl_ref, x_ref, lg_ref):
  x = agg_ref[:N, :] + b_ref[...]
  x_ref[...] = x
  lg_ref[...] = (jnp.dot(x, wl_ref[...], preferred_element_type=jnp.float32)
                 + bl_ref[...])


def _tc_final(agg, b, wl, bl):
  return pl.pallas_call(
      _final_body,
      out_shape=(jax.ShapeDtypeStruct((N, D), jnp.float32),
                 jax.ShapeDtypeStruct((N, L), jnp.float32)),
  )(agg, b, wl, bl)


def kernel(emb, W1, b1, W2, b2, W3, b3, gamma, beta, Wl, bl, edge_index):
  src = edge_index[0].astype(jnp.int32)
  dst = edge_index[1].astype(jnp.int32)
  npad = E_PAD - src.shape[0]
  src_p = jnp.concatenate([src, jnp.zeros((npad,), jnp.int32)])
  dst_p = jnp.concatenate([dst, jnp.full((npad,), -1, jnp.int32)])

  (counts,) = _sc_count(dst_p)
  offs, pr, rounds, starts = _tc_prefix(counts.reshape(NW, 48))
  bsrc, bldst = _sc_bucket(src_p, dst_p, offs.reshape(NW * 32),
                           pr.reshape(NW * 32))
  rounds48 = jnp.pad(rounds, (0, 16))
  starts48 = jnp.pad(starts, (0, 16))

  h1 = _tc_matmul(emb, W1)
  (agg1,) = _sc_agg(bsrc, bldst, h1, rounds48, starts48)
  h2 = _tc_mid(agg1, b1, gamma, beta, W2)
  (agg2,) = _sc_agg(bsrc, bldst, h2, rounds48, starts48)
  h3 = _tc_mid(agg2, b2, gamma, beta, W3)
  (agg3,) = _sc_agg(bsrc, bldst, h3, rounds48, starts48)
  x3, logits = _tc_final(agg3, b3, Wl, bl)
  return (x3, logits)
